# baseline (speedup 1.0000x reference)
import functools

import jax
import jax.numpy as jnp
from jax.experimental import pallas as pl

_MB = 256


def _leaky(x):
    return jnp.where(x >= 0, x, 0.01 * x)


def _relu(x):
    return jnp.maximum(x, 0.0)


def _ident(x):
    return x


def _pad_rows(a, mb=_MB):
    m = a.shape[0]
    mp = ((m + mb - 1) // mb) * mb
    if mp != m:
        a = jnp.pad(a, ((0, mp - m),) + ((0, 0),) * (a.ndim - 1))
    return a, mp


def _mm_body(act, x_ref, w_ref, b_ref, o_ref):
    acc = jnp.dot(x_ref[...], w_ref[...], preferred_element_type=jnp.float32)
    o_ref[...] = act(acc + b_ref[...])


def _mm(x, w, b, act=_ident):
    m, k = x.shape
    n = w.shape[1]
    xp, mp = _pad_rows(x)
    out = pl.pallas_call(
        functools.partial(_mm_body, act),
        grid=(mp // _MB,),
        in_specs=[
            pl.BlockSpec((_MB, k), lambda i: (i, 0)),
            pl.BlockSpec((k, n), lambda i: (0, 0)),
            pl.BlockSpec((1, n), lambda i: (0, 0)),
        ],
        out_specs=pl.BlockSpec((_MB, n), lambda i: (i, 0)),
        out_shape=jax.ShapeDtypeStruct((mp, n), jnp.float32),
    )(xp, w, b.reshape(1, n))
    return out[:m]


def _phi_body(v_ref, w1_ref, b1_ref, w2_ref, b2_ref, o_ref):
    u = v_ref[...]
    w1 = w1_ref[...]
    b1 = b1_ref[...]
    s = _relu(u * w1 + b1) + _relu(-u * w1 + b1)
    o_ref[...] = (
        jnp.dot(s, w2_ref[...], preferred_element_type=jnp.float32)
        + 2.0 * b2_ref[...]
    )


def _phi(v_flat, w1, b1, w2, b2):
    m = v_flat.shape[0]
    h = w2.shape[1]
    xp, mp = _pad_rows(v_flat)
    out = pl.pallas_call(
        _phi_body,
        grid=(mp // _MB,),
        in_specs=[
            pl.BlockSpec((_MB, 1), lambda i: (i, 0)),
            pl.BlockSpec((1, h), lambda i: (0, 0)),
            pl.BlockSpec((1, h), lambda i: (0, 0)),
            pl.BlockSpec((h, h), lambda i: (0, 0)),
            pl.BlockSpec((1, h), lambda i: (0, 0)),
        ],
        out_specs=pl.BlockSpec((_MB, h), lambda i: (i, 0)),
        out_shape=jax.ShapeDtypeStruct((mp, h), jnp.float32),
    )(xp, w1.reshape(1, h), b1.reshape(1, h), w2, b2.reshape(1, h))
    return out[:m]


def _gru_body(x_ref, h_ref, wih_ref, whh_ref, bih_ref, bhh_ref, o_ref):
    hdim = h_ref.shape[1]
    xv = x_ref[...]
    xe = jnp.where(xv >= 0, xv, jnp.exp(jnp.minimum(xv, 0.0)) - 1.0)
    hh = h_ref[...]
    gi = jnp.dot(xe, wih_ref[...], preferred_element_type=jnp.float32) + bih_ref[...]
    gh = jnp.dot(hh, whh_ref[...], preferred_element_type=jnp.float32) + bhh_ref[...]
    ir, iz, inn = gi[:, :hdim], gi[:, hdim : 2 * hdim], gi[:, 2 * hdim :]
    hr, hz, hn = gh[:, :hdim], gh[:, hdim : 2 * hdim], gh[:, 2 * hdim :]
    r = jax.nn.sigmoid(ir + hr)
    z = jax.nn.sigmoid(iz + hz)
    n = jnp.tanh(inn + r * hn)
    o_ref[...] = _relu((1.0 - z) * n + z * hh)


def _gru(m, h, wih, whh, bih, bhh):
    rows, hdim = h.shape
    mp_in, mp = _pad_rows(m)
    hp, _ = _pad_rows(h)
    out = pl.pallas_call(
        _gru_body,
        grid=(mp // _MB,),
        in_specs=[
            pl.BlockSpec((_MB, hdim), lambda i: (i, 0)),
            pl.BlockSpec((_MB, hdim), lambda i: (i, 0)),
            pl.BlockSpec((hdim, 3 * hdim), lambda i: (0, 0)),
            pl.BlockSpec((hdim, 3 * hdim), lambda i: (0, 0)),
            pl.BlockSpec((1, 3 * hdim), lambda i: (0, 0)),
            pl.BlockSpec((1, 3 * hdim), lambda i: (0, 0)),
        ],
        out_specs=pl.BlockSpec((_MB, hdim), lambda i: (i, 0)),
        out_shape=jax.ShapeDtypeStruct((mp, hdim), jnp.float32),
    )(mp_in, hp, wih, whh, bih.reshape(1, -1), bhh.reshape(1, -1))
    return out[:rows]


def _segment_softmax(s, seg, num):
    m = jax.ops.segment_max(s, seg, num_segments=num)
    m = jnp.where(jnp.isfinite(m), m, 0.0)
    e = jnp.exp(s - m[seg])
    d = jax.ops.segment_sum(e, seg, num_segments=num)
    return e / (d[seg] + 1e-16)


def kernel(x, eig_vecs, edge_attr, dist_rbf, edge_index, batch, params):
    p = params
    n_nodes, node_dim = x.shape
    pe = eig_vecs.shape[1]
    hdim = p['phi_w2'].shape[1]
    n_graphs = 128
    n_heads = 8
    head_dim = node_dim // n_heads
    src = edge_index[0]
    dst = edge_index[1]

    z0 = _phi(eig_vecs.reshape(-1, 1), p['phi_w1'], p['phi_b1'],
              p['phi_w2'], p['phi_b2']).reshape(n_nodes, pe, hdim)
    e_h = _mm(edge_attr, p['sn_edge_w'], p['sn_edge_b'], _relu)
    z_sum = z0.sum(axis=1)
    msg = z_sum[src] + e_h
    agg = jax.ops.segment_sum(msg, dst, num_segments=n_nodes)
    z = _relu(z0 + agg[:, None, :])
    pos = _mm(z.reshape(n_nodes, -1), p['rho_w'], p['rho_b'])

    h_in = x + pos
    wqkv = jnp.concatenate([p['Wq'], p['Wk'], p['Wv']], axis=1)
    qkv = _mm(h_in, wqkv, jnp.zeros((3 * node_dim,), jnp.float32))
    q = qkv[:, :node_dim].reshape(n_nodes, n_heads, head_dim)
    k = qkv[:, node_dim:2 * node_dim].reshape(n_nodes, n_heads, head_dim)
    vv = qkv[:, 2 * node_dim:].reshape(n_nodes, n_heads, head_dim)
    sc = (q[dst] * k[src]).sum(-1) / jnp.sqrt(float(head_dim))
    al = _segment_softmax(sc, dst, n_nodes)
    node = jax.ops.segment_sum(al[..., None] * vv[src], dst,
                               num_segments=n_nodes).reshape(n_nodes, node_dim)

    h = _mm(node, p['lin1_w'], p['lin1_b'], _leaky)
    hW1 = _mm(h, p['gate_w1'][:hdim], jnp.zeros((hdim,), jnp.float32))
    rbfW1 = _mm(dist_rbf, p['gate_w1'][hdim:], p['gate_b1'])
    xj = _leaky(hW1[src] + rbfW1)
    a = _leaky(xj @ p['gate_att_l'] + (h @ p['gate_att_r'])[dst])
    a = _segment_softmax(a, dst, n_nodes)
    hw2 = _mm(h, p['gate_w2'], jnp.zeros((hdim,), jnp.float32))
    m = jax.ops.segment_sum(a[:, None] * hw2[src], dst, num_segments=n_nodes)
    h = _gru(m, h, p['gru1_wih'], p['gru1_whh'], p['gru1_bih'], p['gru1_bhh'])

    for l in (2, 3):
        hw = _mm(h, p['gat%d_w' % l], jnp.zeros((hdim,), jnp.float32))
        a = _leaky((hw @ p['gat%d_att_src' % l])[src]
                   + (hw @ p['gat%d_att_dst' % l])[dst])
        a = _segment_softmax(a, dst, n_nodes)
        m = jax.ops.segment_sum(a[:, None] * hw[src], dst, num_segments=n_nodes)
        h = _gru(m, h, p['gru%d_wih' % l], p['gru%d_whh' % l],
                 p['gru%d_bih' % l], p['gru%d_bhh' % l])

    g = _relu(jax.ops.segment_sum(h, batch, num_segments=n_graphs))
    hw = _mm(h, p['mol_w'], jnp.zeros((hdim,), jnp.float32))
    hw_src = hw @ p['mol_att_src']
    for _ in range(3):
        gw = _mm(g, p['mol_w'], jnp.zeros((hdim,), jnp.float32))
        a = _leaky(hw_src + (gw @ p['mol_att_dst'])[batch])
        a = _segment_softmax(a, batch, n_graphs)
        hg = jax.ops.segment_sum(a[:, None] * hw, batch, num_segments=n_graphs)
        g = _gru(hg, g, p['molgru_wih'], p['molgru_whh'],
                 p['molgru_bih'], p['molgru_bhh'])

    emb = _mm(g, p['lin2_w'], p['lin2_b'])

    r1 = _mm(emb, p['reg_w1'], p['reg_b1'], _relu)
    w2p = jnp.pad(p['reg_w2'], ((0, 0), (0, 127)))
    b2p = jnp.pad(p['reg_b2'], (0, 127))
    out = _mm(r1, w2p, b2p)[:, :1]
    return out


# speedup vs baseline: 1.0104x; 1.0104x over previous
import functools

import jax
import jax.numpy as jnp
from jax.experimental import pallas as pl

_MB = 256


def _leaky(x):
    return jnp.where(x >= 0, x, 0.01 * x)


def _relu(x):
    return jnp.maximum(x, 0.0)


def _ident(x):
    return x


def _tile_rows(m):
    for mb in (256, 512, 400, 200, 128, 80):
        if m % mb == 0:
            return mb, m
    return _MB, ((m + _MB - 1) // _MB) * _MB


def _pad_rows(a, mp):
    m = a.shape[0]
    if mp != m:
        a = jnp.pad(a, ((0, mp - m),) + ((0, 0),) * (a.ndim - 1))
    return a


def _mm_body(act, x_ref, w_ref, b_ref, o_ref):
    acc = jnp.dot(x_ref[...], w_ref[...], preferred_element_type=jnp.float32)
    o_ref[...] = act(acc + b_ref[...])


def _mm(x, w, b, act=_ident):
    m, k = x.shape
    n = w.shape[1]
    mb, mp = _tile_rows(m)
    xp = _pad_rows(x, mp)
    out = pl.pallas_call(
        functools.partial(_mm_body, act),
        grid=(mp // mb,),
        in_specs=[
            pl.BlockSpec((mb, k), lambda i: (i, 0)),
            pl.BlockSpec((k, n), lambda i: (0, 0)),
            pl.BlockSpec((1, n), lambda i: (0, 0)),
        ],
        out_specs=pl.BlockSpec((mb, n), lambda i: (i, 0)),
        out_shape=jax.ShapeDtypeStruct((mp, n), jnp.float32),
    )(xp, w, b.reshape(1, n))
    return out[:m]


def _phi_body(v_ref, w1_ref, b1_ref, w2_ref, b2_ref, o_ref):
    u = v_ref[...]
    w1 = w1_ref[...]
    b1 = b1_ref[...]
    s = _relu(u * w1 + b1) + _relu(-u * w1 + b1)
    o_ref[...] = (
        jnp.dot(s, w2_ref[...], preferred_element_type=jnp.float32)
        + 2.0 * b2_ref[...]
    )


def _phi(v_flat, w1, b1, w2, b2):
    m = v_flat.shape[0]
    h = w2.shape[1]
    mb, mp = _tile_rows(m)
    xp = _pad_rows(v_flat, mp)
    out = pl.pallas_call(
        _phi_body,
        grid=(mp // mb,),
        in_specs=[
            pl.BlockSpec((mb, 1), lambda i: (i, 0)),
            pl.BlockSpec((1, h), lambda i: (0, 0)),
            pl.BlockSpec((1, h), lambda i: (0, 0)),
            pl.BlockSpec((h, h), lambda i: (0, 0)),
            pl.BlockSpec((1, h), lambda i: (0, 0)),
        ],
        out_specs=pl.BlockSpec((mb, h), lambda i: (i, 0)),
        out_shape=jax.ShapeDtypeStruct((mp, h), jnp.float32),
    )(xp, w1.reshape(1, h), b1.reshape(1, h), w2, b2.reshape(1, h))
    return out[:m]


def _gru_body(x_ref, h_ref, wih_ref, whh_ref, bih_ref, bhh_ref, o_ref):
    hdim = h_ref.shape[1]
    xv = x_ref[...]
    xe = jnp.where(xv >= 0, xv, jnp.exp(jnp.minimum(xv, 0.0)) - 1.0)
    hh = h_ref[...]
    gi = jnp.dot(xe, wih_ref[...], preferred_element_type=jnp.float32) + bih_ref[...]
    gh = jnp.dot(hh, whh_ref[...], preferred_element_type=jnp.float32) + bhh_ref[...]
    ir, iz, inn = gi[:, :hdim], gi[:, hdim : 2 * hdim], gi[:, 2 * hdim :]
    hr, hz, hn = gh[:, :hdim], gh[:, hdim : 2 * hdim], gh[:, 2 * hdim :]
    r = jax.nn.sigmoid(ir + hr)
    z = jax.nn.sigmoid(iz + hz)
    n = jnp.tanh(inn + r * hn)
    o_ref[...] = _relu((1.0 - z) * n + z * hh)


def _gru(m, h, wih, whh, bih, bhh):
    rows, hdim = h.shape
    mb, mp = _tile_rows(rows)
    mp_in = _pad_rows(m, mp)
    hp = _pad_rows(h, mp)
    out = pl.pallas_call(
        _gru_body,
        grid=(mp // mb,),
        in_specs=[
            pl.BlockSpec((mb, hdim), lambda i: (i, 0)),
            pl.BlockSpec((mb, hdim), lambda i: (i, 0)),
            pl.BlockSpec((hdim, 3 * hdim), lambda i: (0, 0)),
            pl.BlockSpec((hdim, 3 * hdim), lambda i: (0, 0)),
            pl.BlockSpec((1, 3 * hdim), lambda i: (0, 0)),
            pl.BlockSpec((1, 3 * hdim), lambda i: (0, 0)),
        ],
        out_specs=pl.BlockSpec((mb, hdim), lambda i: (i, 0)),
        out_shape=jax.ShapeDtypeStruct((mp, hdim), jnp.float32),
    )(mp_in, hp, wih, whh, bih.reshape(1, -1), bhh.reshape(1, -1))
    return out[:rows]


def _segment_softmax(s, seg, num):
    m = jax.ops.segment_max(s, seg, num_segments=num)
    m = jnp.where(jnp.isfinite(m), m, 0.0)
    e = jnp.exp(s - m[seg])
    d = jax.ops.segment_sum(e, seg, num_segments=num)
    return e / (d[seg] + 1e-16)


def kernel(x, eig_vecs, edge_attr, dist_rbf, edge_index, batch, params):
    p = params
    n_nodes, node_dim = x.shape
    pe = eig_vecs.shape[1]
    hdim = p['phi_w2'].shape[1]
    n_graphs = 128
    n_heads = 8
    head_dim = node_dim // n_heads
    src = edge_index[0]
    dst = edge_index[1]

    z0 = _phi(eig_vecs.reshape(-1, 1), p['phi_w1'], p['phi_b1'],
              p['phi_w2'], p['phi_b2']).reshape(n_nodes, pe, hdim)
    e_h = _mm(edge_attr, p['sn_edge_w'], p['sn_edge_b'], _relu)
    z_sum = z0.sum(axis=1)
    msg = z_sum[src] + e_h
    agg = jax.ops.segment_sum(msg, dst, num_segments=n_nodes)
    z = _relu(z0 + agg[:, None, :])
    pos = _mm(z.reshape(n_nodes, -1), p['rho_w'], p['rho_b'])

    h_in = x + pos
    wqkv = jnp.concatenate([p['Wq'], p['Wk'], p['Wv']], axis=1)
    qkv = _mm(h_in, wqkv, jnp.zeros((3 * node_dim,), jnp.float32))
    q = qkv[:, :node_dim].reshape(n_nodes, n_heads, head_dim)
    k = qkv[:, node_dim:2 * node_dim].reshape(n_nodes, n_heads, head_dim)
    vv = qkv[:, 2 * node_dim:].reshape(n_nodes, n_heads, head_dim)
    sc = (q[dst] * k[src]).sum(-1) / jnp.sqrt(float(head_dim))
    al = _segment_softmax(sc, dst, n_nodes)
    node = jax.ops.segment_sum(al[..., None] * vv[src], dst,
                               num_segments=n_nodes).reshape(n_nodes, node_dim)

    h = _mm(node, p['lin1_w'], p['lin1_b'], _leaky)
    w_gate = jnp.concatenate(
        [p['gate_w1'][:hdim], p['gate_w2'],
         jnp.pad(p['gate_att_r'][:, None], ((0, 0), (0, hdim - 1)))], axis=1)
    gproj = _mm(h, w_gate, jnp.zeros((3 * hdim,), jnp.float32))
    hW1 = gproj[:, :hdim]
    hw2 = gproj[:, hdim:2 * hdim]
    h_att_r = gproj[:, 2 * hdim]
    rbfW1 = _mm(dist_rbf, p['gate_w1'][hdim:], p['gate_b1'])
    xj = _leaky(hW1[src] + rbfW1)
    a = _leaky(xj @ p['gate_att_l'] + h_att_r[dst])
    a = _segment_softmax(a, dst, n_nodes)
    m = jax.ops.segment_sum(a[:, None] * hw2[src], dst, num_segments=n_nodes)
    h = _gru(m, h, p['gru1_wih'], p['gru1_whh'], p['gru1_bih'], p['gru1_bhh'])

    for l in (2, 3):
        hw = _mm(h, p['gat%d_w' % l], jnp.zeros((hdim,), jnp.float32))
        a = _leaky((hw @ p['gat%d_att_src' % l])[src]
                   + (hw @ p['gat%d_att_dst' % l])[dst])
        a = _segment_softmax(a, dst, n_nodes)
        m = jax.ops.segment_sum(a[:, None] * hw[src], dst, num_segments=n_nodes)
        h = _gru(m, h, p['gru%d_wih' % l], p['gru%d_whh' % l],
                 p['gru%d_bih' % l], p['gru%d_bhh' % l])

    g = _relu(jax.ops.segment_sum(h, batch, num_segments=n_graphs))
    hw = _mm(h, p['mol_w'], jnp.zeros((hdim,), jnp.float32))
    hw_src = hw @ p['mol_att_src']
    for _ in range(3):
        gw = _mm(g, p['mol_w'], jnp.zeros((hdim,), jnp.float32))
        a = _leaky(hw_src + (gw @ p['mol_att_dst'])[batch])
        a = _segment_softmax(a, batch, n_graphs)
        hg = jax.ops.segment_sum(a[:, None] * hw, batch, num_segments=n_graphs)
        g = _gru(hg, g, p['molgru_wih'], p['molgru_whh'],
                 p['molgru_bih'], p['molgru_bhh'])

    emb = _mm(g, p['lin2_w'], p['lin2_b'])

    r1 = _mm(emb, p['reg_w1'], p['reg_b1'], _relu)
    w2p = jnp.pad(p['reg_w2'], ((0, 0), (0, 127)))
    b2p = jnp.pad(p['reg_b2'], (0, 127))
    out = _mm(r1, w2p, b2p)[:, :1]
    return out
